# Initial kernel scaffold; baseline (speedup 1.0000x reference)
import functools

import jax
import jax.numpy as jnp
from jax import lax
from jax.experimental import pallas as pl
from jax.experimental.pallas import tpu as pltpu
from jax.experimental.pallas import tpu_sc as plsc

N = 10000
E = 320000
C = 128
NC = 2
NS = 16
NW = NC * NS
EPW = E // NW
B = 80
NCHUNK = EPW // B
ROWS_PER_TILE = N // NS
ZROWS = 125

_R = 400


def _tc_pre_body(x_ref, w_ref, a2_ref, h_ref, a_ref):
    h = jnp.dot(x_ref[...], w_ref[...], precision=lax.Precision.HIGHEST,
                preferred_element_type=jnp.float32)
    h_ref[...] = h
    a_ref[...] = jnp.dot(h, a2_ref[...], precision=lax.Precision.HIGHEST,
                         preferred_element_type=jnp.float32)


def _tc_pre(x, W, a2):
    grid = (N // _R,)
    return pl.pallas_call(
        _tc_pre_body,
        grid=grid,
        in_specs=[
            pl.BlockSpec((_R, C), lambda i: (i, 0)),
            pl.BlockSpec((C, C), lambda i: (0, 0)),
            pl.BlockSpec((C, C), lambda i: (0, 0)),
        ],
        out_specs=[
            pl.BlockSpec((_R, C), lambda i: (i, 0)),
            pl.BlockSpec((_R, C), lambda i: (i, 0)),
        ],
        out_shape=[
            jax.ShapeDtypeStruct((N, C), jnp.float32),
            jax.ShapeDtypeStruct((N, C), jnp.float32),
        ],
    )(x, W, a2)


def _sc_edges(asrc, adst, h, src, dst):
    mesh = plsc.VectorSubcoreMesh(core_axis_name="c", subcore_axis_name="s")

    @functools.partial(
        pl.kernel,
        out_type=[
            jax.ShapeDtypeStruct((NC, N, C), jnp.float32),
            jax.ShapeDtypeStruct((NC, N, 16), jnp.float32),
        ],
        mesh=mesh,
        scratch_types=[
            pltpu.VMEM((N,), jnp.float32),
            pltpu.VMEM((N,), jnp.float32),
            pltpu.VMEM((B,), jnp.int32),
            pltpu.VMEM((B,), jnp.int32),
            pltpu.VMEM((B, C), jnp.float32),
            pltpu.VMEM((B,), jnp.float32),
            pltpu.VMEM((B, 16), jnp.float32),
            pltpu.VMEM((ZROWS, C), jnp.float32),
            pltpu.VMEM((ZROWS, 16), jnp.float32),
            pltpu.VMEM_SHARED((N, C), jnp.float32),
            pltpu.VMEM_SHARED((N, 16), jnp.float32),
            pltpu.SemaphoreType.DMA,
        ],
    )
    def k(asrc_hbm, adst_hbm, h_hbm, src_hbm, dst_hbm, acc_hbm, den_hbm,
          asrc_v, adst_v, srcv, dstv, rows, sv, denst, zacc, zden,
          acc_sh, den_sh, sem):
        cid = lax.axis_index("c")
        sid = lax.axis_index("s")
        wid = sid * NC + cid

        zero16 = jnp.zeros((16,), jnp.float32)

        @pl.loop(0, ZROWS)
        def _(r):
            for cc in range(C // 16):
                zacc[r, pl.ds(cc * 16, 16)] = zero16
            zden[r, :] = zero16

        @pl.loop(0, ROWS_PER_TILE // ZROWS)
        def _(z):
            base = sid * ROWS_PER_TILE + z * ZROWS
            pltpu.sync_copy(zacc, acc_sh.at[pl.ds(base, ZROWS)])
            pltpu.sync_copy(zden, den_sh.at[pl.ds(base, ZROWS)])

        plsc.subcore_barrier()

        pltpu.sync_copy(asrc_hbm, asrc_v)
        pltpu.sync_copy(adst_hbm, adst_v)

        @pl.loop(0, NCHUNK)
        def _(g):
            row = wid * NCHUNK + g
            pltpu.sync_copy(src_hbm.at[row], srcv)
            pltpu.sync_copy(dst_hbm.at[row], dstv)
            pltpu.async_copy(h_hbm.at[srcv], rows, sem).wait()

            @pl.loop(0, B, step=16)
            def _(i):
                si = srcv[pl.ds(i, 16)]
                di = dstv[pl.ds(i, 16)]
                al = plsc.load_gather(asrc_v, [si]) + plsc.load_gather(adst_v, [di])
                al = jnp.where(al >= 0.0, al, al * 0.2)
                sv[pl.ds(i, 16)] = jnp.exp(al)

            @pl.loop(0, B)
            def _(i):
                spl = plsc.load_gather(sv, [jnp.full((16,), i, jnp.int32)])
                for cc in range(C // 16):
                    sl = pl.ds(cc * 16, 16)
                    rows[i, sl] = rows[i, sl] * spl
                denst[i, :] = spl

            pltpu.sync_copy(rows, acc_sh.at[dstv], add=True)
            pltpu.sync_copy(denst, den_sh.at[dstv], add=True)

        plsc.subcore_barrier()

        base = sid * ROWS_PER_TILE
        pltpu.sync_copy(acc_sh.at[pl.ds(base, ROWS_PER_TILE)],
                        acc_hbm.at[cid, pl.ds(base, ROWS_PER_TILE)])
        pltpu.sync_copy(den_sh.at[pl.ds(base, ROWS_PER_TILE)],
                        den_hbm.at[cid, pl.ds(base, ROWS_PER_TILE)])

    return k(asrc, adst, h, src, dst)


def _tc_post_body(acc0, acc1, den0, den1, h_ref, a_ref, b_ref, out_ref):
    al = a_ref[:, 0:1] + a_ref[:, 1:2]
    al = jnp.where(al >= 0.0, al, al * 0.2)
    sii = jnp.exp(al)
    num = acc0[...] + acc1[...] + sii * h_ref[...]
    den = den0[:, 0:1] + den1[:, 0:1] + sii + 1e-16
    o = num / den + b_ref[...]
    out_ref[...] = jnp.where(o > 0.0, o, jnp.expm1(o))


def _tc_post(acc, den, h, a, bias):
    grid = (N // _R,)
    return pl.pallas_call(
        _tc_post_body,
        grid=grid,
        in_specs=[
            pl.BlockSpec((_R, C), lambda i: (i, 0)),
            pl.BlockSpec((_R, C), lambda i: (i, 0)),
            pl.BlockSpec((_R, 16), lambda i: (i, 0)),
            pl.BlockSpec((_R, 16), lambda i: (i, 0)),
            pl.BlockSpec((_R, C), lambda i: (i, 0)),
            pl.BlockSpec((_R, C), lambda i: (i, 0)),
            pl.BlockSpec((1, C), lambda i: (0, 0)),
        ],
        out_specs=pl.BlockSpec((_R, C), lambda i: (i, 0)),
        out_shape=jax.ShapeDtypeStruct((N, C), jnp.float32),
    )(acc[0], acc[1], den[0], den[1], h, a, bias)


def kernel(x, edge_index, W, att_src, att_dst, bias):
    a2 = jnp.zeros((C, C), jnp.float32)
    a2 = a2.at[:, 0].set(att_src[0]).at[:, 1].set(att_dst[0])
    h, a = _tc_pre(x, W, a2)

    asrc = a[:, 0]
    adst = a[:, 1]
    src = edge_index[0].reshape(NW * NCHUNK, B)
    dst = edge_index[1].reshape(NW * NCHUNK, B)

    acc, den = _sc_edges(asrc, adst, h, src, dst)
    return _tc_post(acc, den, h, a, bias.reshape(1, C))


# speedup vs baseline: 13.9771x; 13.9771x over previous
import dataclasses
import functools

import jax
import jax.numpy as jnp
from jax import lax
from jax.experimental import pallas as pl
from jax.experimental.pallas import tpu as pltpu
from jax.experimental.pallas import tpu_sc as plsc

N = 10000
E = 320000
C = 128
NC = 2
NS = 16
NW = NC * NS
EPW = E // NW
B = 80
NCHUNK = EPW // B
ROWS_PER_TILE = 624
TAIL_BASE = ROWS_PER_TILE * NS
TAIL = N - TAIL_BASE
ZROWS = 104

_R = 400


def _tc_pre_body(x_ref, w_ref, a2_ref, h_ref, a_ref):
    h = jnp.dot(x_ref[...], w_ref[...], precision=lax.Precision.HIGHEST,
                preferred_element_type=jnp.float32)
    h_ref[...] = h
    a_ref[...] = jnp.dot(h, a2_ref[...], precision=lax.Precision.HIGHEST,
                         preferred_element_type=jnp.float32)


def _tc_pre(x, W, a2):
    grid = (N // _R,)
    return pl.pallas_call(
        _tc_pre_body,
        grid=grid,
        in_specs=[
            pl.BlockSpec((_R, C), lambda i: (i, 0)),
            pl.BlockSpec((C, C), lambda i: (0, 0)),
            pl.BlockSpec((C, C), lambda i: (0, 0)),
        ],
        out_specs=[
            pl.BlockSpec((_R, C), lambda i: (i, 0)),
            pl.BlockSpec((_R, C), lambda i: (i, 0)),
        ],
        out_shape=[
            jax.ShapeDtypeStruct((N, C), jnp.float32),
            jax.ShapeDtypeStruct((N, C), jnp.float32),
        ],
    )(x, W, a2)


def _sc_mesh_and_params():
    mesh = plsc.VectorSubcoreMesh(core_axis_name="c", subcore_axis_name="s")
    cp = pltpu.CompilerParams()
    if "needs_layout_passes" in pltpu.CompilerParams.__dataclass_fields__:
        cp = dataclasses.replace(cp, needs_layout_passes=False)
    return mesh, cp


def _sc_logits(asrc, adst, src, dst):
    mesh, cp = _sc_mesh_and_params()

    @functools.partial(
        pl.kernel,
        compiler_params=cp,
        out_type=[
            jax.ShapeDtypeStruct((NW * NCHUNK, 1, B), jnp.float32),
            jax.ShapeDtypeStruct((NC, N, C), jnp.float32),
        ],
        mesh=mesh,
        scratch_types=[
            pltpu.VMEM((N,), jnp.float32),
            pltpu.VMEM((N,), jnp.float32),
            pltpu.VMEM((1, B), jnp.int32),
            pltpu.VMEM((1, B), jnp.int32),
            pltpu.VMEM((1, B), jnp.float32),
            pltpu.VMEM((B, C), jnp.float32),
            pltpu.VMEM_SHARED((N, C), jnp.float32),
        ],
    )
    def k(asrc_hbm, adst_hbm, src_hbm, dst_hbm, s_hbm, den_hbm,
          asrc_v, adst_v, srcv, dstv, sv, denst, den_sh):
        cid = lax.axis_index("c")
        sid = lax.axis_index("s")
        wid = sid * NC + cid

        zero16 = jnp.zeros((16,), jnp.float32)

        @pl.loop(0, B)
        def _(r):
            for cc in range(C // 16):
                denst[r, pl.ds(cc * 16, 16)] = zero16

        @pl.loop(0, ROWS_PER_TILE // B)
        def _(z):
            base = sid * ROWS_PER_TILE + z * B
            pltpu.sync_copy(denst, den_sh.at[pl.ds(base, B)])
        pltpu.sync_copy(denst.at[pl.ds(0, ROWS_PER_TILE % B)],
                        den_sh.at[pl.ds(sid * ROWS_PER_TILE
                                        + (ROWS_PER_TILE // B) * B,
                                        ROWS_PER_TILE % B)])

        @pl.when(sid == NS - 1)
        def _():
            pltpu.sync_copy(denst.at[pl.ds(0, TAIL)],
                            den_sh.at[pl.ds(TAIL_BASE, TAIL)])

        pltpu.sync_copy(asrc_hbm, asrc_v)
        pltpu.sync_copy(adst_hbm, adst_v)

        plsc.subcore_barrier()

        @pl.loop(0, NCHUNK)
        def _(g):
            row = wid * NCHUNK + g
            pltpu.sync_copy(src_hbm.at[row], srcv)
            pltpu.sync_copy(dst_hbm.at[row], dstv)

            @pl.loop(0, B, step=16)
            def _(i):
                si = srcv[0, pl.ds(i, 16)]
                di = dstv[0, pl.ds(i, 16)]
                al = plsc.load_gather(asrc_v, [si]) + plsc.load_gather(adst_v, [di])
                al = jnp.where(al >= 0.0, al, al * 0.2)
                sv[0, pl.ds(i, 16)] = jnp.exp(al)

            pltpu.sync_copy(sv, s_hbm.at[row])

            zi16 = jnp.zeros((16,), jnp.int32)

            @pl.loop(0, B)
            def _(i):
                spl = plsc.load_gather(sv, [zi16, jnp.full((16,), i, jnp.int32)])
                for cc in range(C // 16):
                    denst[i, pl.ds(cc * 16, 16)] = spl

            pltpu.sync_copy(denst, den_sh.at[dstv.at[0]], add=True)

        plsc.subcore_barrier()

        base = sid * ROWS_PER_TILE
        pltpu.sync_copy(den_sh.at[pl.ds(base, ROWS_PER_TILE)],
                        den_hbm.at[cid, pl.ds(base, ROWS_PER_TILE)])

        @pl.when(sid == NS - 1)
        def _():
            pltpu.sync_copy(den_sh.at[pl.ds(TAIL_BASE, TAIL)],
                            den_hbm.at[cid, pl.ds(TAIL_BASE, TAIL)])

    return k(asrc, adst, src, dst)


def _sc_messages(h, src, dst, s):
    mesh, cp = _sc_mesh_and_params()

    @functools.partial(
        pl.kernel,
        compiler_params=cp,
        out_type=jax.ShapeDtypeStruct((NC, N, C), jnp.float32),
        mesh=mesh,
        scratch_types=[
            pltpu.VMEM((1, B), jnp.int32),
            pltpu.VMEM((1, B), jnp.int32),
            pltpu.VMEM((1, B), jnp.float32),
            pltpu.VMEM((B, C), jnp.float32),
            pltpu.VMEM_SHARED((N, C), jnp.float32),
            pltpu.SemaphoreType.DMA,
        ],
    )
    def k(h_hbm, src_hbm, dst_hbm, s_hbm, acc_hbm,
          srcv, dstv, sv, rows, acc_sh, sem):
        cid = lax.axis_index("c")
        sid = lax.axis_index("s")
        wid = sid * NC + cid

        zero16 = jnp.zeros((16,), jnp.float32)

        @pl.loop(0, B)
        def _(r):
            for cc in range(C // 16):
                rows[r, pl.ds(cc * 16, 16)] = zero16

        @pl.loop(0, ROWS_PER_TILE // B)
        def _(z):
            base = sid * ROWS_PER_TILE + z * B
            pltpu.sync_copy(rows, acc_sh.at[pl.ds(base, B)])
        pltpu.sync_copy(rows.at[pl.ds(0, ROWS_PER_TILE % B)],
                        acc_sh.at[pl.ds(sid * ROWS_PER_TILE
                                        + (ROWS_PER_TILE // B) * B,
                                        ROWS_PER_TILE % B)])

        @pl.when(sid == NS - 1)
        def _():
            pltpu.sync_copy(rows.at[pl.ds(0, TAIL)],
                            acc_sh.at[pl.ds(TAIL_BASE, TAIL)])

        plsc.subcore_barrier()

        zi16 = jnp.zeros((16,), jnp.int32)

        @pl.loop(0, NCHUNK)
        def _(g):
            row = wid * NCHUNK + g
            pltpu.sync_copy(src_hbm.at[row], srcv)
            pltpu.sync_copy(dst_hbm.at[row], dstv)
            pltpu.sync_copy(s_hbm.at[row], sv)
            pltpu.async_copy(h_hbm.at[srcv.at[0]], rows, sem).wait()

            @pl.loop(0, B)
            def _(i):
                spl = plsc.load_gather(sv, [zi16, jnp.full((16,), i, jnp.int32)])
                for cc in range(C // 16):
                    sl = pl.ds(cc * 16, 16)
                    rows[i, sl] = rows[i, sl] * spl

            pltpu.sync_copy(rows, acc_sh.at[dstv.at[0]], add=True)

        plsc.subcore_barrier()

        base = sid * ROWS_PER_TILE
        pltpu.sync_copy(acc_sh.at[pl.ds(base, ROWS_PER_TILE)],
                        acc_hbm.at[cid, pl.ds(base, ROWS_PER_TILE)])

        @pl.when(sid == NS - 1)
        def _():
            pltpu.sync_copy(acc_sh.at[pl.ds(TAIL_BASE, TAIL)],
                            acc_hbm.at[cid, pl.ds(TAIL_BASE, TAIL)])

    return k(h, src, dst, s)


def _tc_post_body(acc0, acc1, den0, den1, h_ref, a_ref, b_ref, out_ref):
    al = a_ref[:, 0:1] + a_ref[:, 1:2]
    al = jnp.where(al >= 0.0, al, al * 0.2)
    sii = jnp.exp(al)
    num = acc0[...] + acc1[...] + sii * h_ref[...]
    den = den0[:, 0:1] + den1[:, 0:1] + sii + 1e-16
    o = num / den + b_ref[...]
    out_ref[...] = jnp.where(o > 0.0, o, jnp.exp(o) - 1.0)


def _tc_post(acc, den, h, a, bias):
    grid = (N // _R,)
    return pl.pallas_call(
        _tc_post_body,
        grid=grid,
        in_specs=[
            pl.BlockSpec((_R, C), lambda i: (i, 0)),
            pl.BlockSpec((_R, C), lambda i: (i, 0)),
            pl.BlockSpec((_R, C), lambda i: (i, 0)),
            pl.BlockSpec((_R, C), lambda i: (i, 0)),
            pl.BlockSpec((_R, C), lambda i: (i, 0)),
            pl.BlockSpec((_R, C), lambda i: (i, 0)),
            pl.BlockSpec((1, C), lambda i: (0, 0)),
        ],
        out_specs=pl.BlockSpec((_R, C), lambda i: (i, 0)),
        out_shape=jax.ShapeDtypeStruct((N, C), jnp.float32),
    )(acc[0], acc[1], den[0], den[1], h, a, bias)


def kernel(x, edge_index, W, att_src, att_dst, bias):
    a2 = jnp.zeros((C, C), jnp.float32)
    a2 = a2.at[:, 0].set(att_src[0]).at[:, 1].set(att_dst[0])
    h, a = _tc_pre(x, W, a2)

    asrc = a[:, 0]
    adst = a[:, 1]
    src = edge_index[0].reshape(NW * NCHUNK, 1, B)
    dst = edge_index[1].reshape(NW * NCHUNK, 1, B)

    s, den = _sc_logits(asrc, adst, src, dst)
    acc = _sc_messages(h, src, dst, s)
    return _tc_post(acc, den, h, a, bias.reshape(1, C))


# speedup vs baseline: 16.7578x; 1.1990x over previous
import dataclasses
import functools

import jax
import jax.numpy as jnp
from jax import lax
from jax.experimental import pallas as pl
from jax.experimental.pallas import tpu as pltpu
from jax.experimental.pallas import tpu_sc as plsc

N = 10000
E = 320000
C = 128
NC = 2
NS = 16
NW = NC * NS
EPW = E // NW
B = 80
NCHUNK = EPW // B
ROWS_PER_TILE = 624
TAIL_BASE = ROWS_PER_TILE * NS
TAIL = N - TAIL_BASE

_R = 400


def _tc_pre_body(x_ref, w_ref, a2_ref, h_ref, a_ref):
    h = jnp.dot(x_ref[...], w_ref[...], precision=lax.Precision.HIGHEST,
                preferred_element_type=jnp.float32)
    h_ref[...] = h
    a_ref[...] = jnp.dot(h, a2_ref[...], precision=lax.Precision.HIGHEST,
                         preferred_element_type=jnp.float32)


def _tc_pre(x, W, a2):
    grid = (N // _R,)
    return pl.pallas_call(
        _tc_pre_body,
        grid=grid,
        in_specs=[
            pl.BlockSpec((_R, C), lambda i: (i, 0)),
            pl.BlockSpec((C, C), lambda i: (0, 0)),
            pl.BlockSpec((C, C), lambda i: (0, 0)),
        ],
        out_specs=[
            pl.BlockSpec((_R, C), lambda i: (i, 0)),
            pl.BlockSpec((_R, C), lambda i: (i, 0)),
        ],
        out_shape=[
            jax.ShapeDtypeStruct((N, C), jnp.float32),
            jax.ShapeDtypeStruct((N, C), jnp.float32),
        ],
    )(x, W, a2)


def _sc_mesh_and_params():
    mesh = plsc.VectorSubcoreMesh(core_axis_name="c", subcore_axis_name="s")
    cp = pltpu.CompilerParams()
    if "needs_layout_passes" in pltpu.CompilerParams.__dataclass_fields__:
        cp = dataclasses.replace(cp, needs_layout_passes=False)
    return mesh, cp


def _sc_logits(asrc, adst, src, dst):
    mesh, cp = _sc_mesh_and_params()

    @functools.partial(
        pl.kernel,
        compiler_params=cp,
        out_type=[
            jax.ShapeDtypeStruct((NW * NCHUNK, 1, B), jnp.float32),
            jax.ShapeDtypeStruct((NW, 1, N), jnp.float32),
        ],
        mesh=mesh,
        scratch_types=[
            pltpu.VMEM((N,), jnp.float32),
            pltpu.VMEM((N,), jnp.float32),
            pltpu.VMEM((1, N), jnp.float32),
            pltpu.VMEM((1, B), jnp.int32),
            pltpu.VMEM((1, B), jnp.int32),
            pltpu.VMEM((1, B), jnp.float32),
        ],
    )
    def k(asrc_hbm, adst_hbm, src_hbm, dst_hbm, s_hbm, den_hbm,
          asrc_v, adst_v, den_v, srcv, dstv, sv):
        cid = lax.axis_index("c")
        sid = lax.axis_index("s")
        wid = sid * NC + cid

        zero16 = jnp.zeros((16,), jnp.float32)

        zi16 = jnp.zeros((16,), jnp.int32)

        @pl.loop(0, N, step=16)
        def _(r):
            den_v[0, pl.ds(r, 16)] = zero16

        pltpu.sync_copy(asrc_hbm, asrc_v)
        pltpu.sync_copy(adst_hbm, adst_v)

        @pl.loop(0, NCHUNK)
        def _(g):
            row = wid * NCHUNK + g
            pltpu.sync_copy(src_hbm.at[row], srcv)
            pltpu.sync_copy(dst_hbm.at[row], dstv)

            @pl.loop(0, B, step=16)
            def _(i):
                si = srcv[0, pl.ds(i, 16)]
                di = dstv[0, pl.ds(i, 16)]
                al = plsc.load_gather(asrc_v, [si]) + plsc.load_gather(adst_v, [di])
                al = jnp.where(al >= 0.0, al, al * 0.2)
                sval = jnp.exp(al)
                sv[0, pl.ds(i, 16)] = sval
                plsc.addupdate_scatter(den_v, [zi16, di], sval)

            pltpu.sync_copy(sv, s_hbm.at[row])

        pltpu.sync_copy(den_v, den_hbm.at[wid])

    return k(asrc, adst, src, dst)


def _sc_messages(h, src, dst, s):
    mesh, cp = _sc_mesh_and_params()

    @functools.partial(
        pl.kernel,
        compiler_params=cp,
        out_type=jax.ShapeDtypeStruct((NC, N, C), jnp.float32),
        mesh=mesh,
        scratch_types=[
            pltpu.VMEM((1, B), jnp.int32),
            pltpu.VMEM((1, B), jnp.int32),
            pltpu.VMEM((1, B), jnp.float32),
            pltpu.VMEM((B, C), jnp.float32),
            pltpu.VMEM_SHARED((N, C), jnp.float32),
            pltpu.SemaphoreType.DMA,
        ],
    )
    def k(h_hbm, src_hbm, dst_hbm, s_hbm, acc_hbm,
          srcv, dstv, sv, rows, acc_sh, sem):
        cid = lax.axis_index("c")
        sid = lax.axis_index("s")
        wid = sid * NC + cid

        zero16 = jnp.zeros((16,), jnp.float32)

        ZB = 80

        @pl.loop(0, ZB)
        def _(r):
            for cc in range(C // 16):
                rows[r, pl.ds(cc * 16, 16)] = zero16

        @pl.loop(0, ROWS_PER_TILE // ZB)
        def _(z):
            base = pl.multiple_of(sid * ROWS_PER_TILE + z * ZB, 8)
            pltpu.sync_copy(rows.at[pl.ds(0, ZB)], acc_sh.at[pl.ds(base, ZB)])
        pltpu.sync_copy(rows.at[pl.ds(0, ROWS_PER_TILE % ZB)],
                        acc_sh.at[pl.ds(pl.multiple_of(
                            sid * ROWS_PER_TILE + (ROWS_PER_TILE // ZB) * ZB, 8),
                            ROWS_PER_TILE % ZB)])

        @pl.when(sid == NS - 1)
        def _():
            pltpu.sync_copy(rows.at[pl.ds(0, TAIL)],
                            acc_sh.at[pl.ds(TAIL_BASE, TAIL)])

        plsc.subcore_barrier()

        zi16 = jnp.zeros((16,), jnp.int32)

        @pl.loop(0, NCHUNK)
        def _(g):
            row = wid * NCHUNK + g
            pltpu.sync_copy(src_hbm.at[row], srcv)
            pltpu.sync_copy(dst_hbm.at[row], dstv)
            pltpu.sync_copy(s_hbm.at[row], sv)
            pltpu.async_copy(h_hbm.at[srcv.at[0]], rows, sem).wait()

            @pl.loop(0, B)
            def _(i):
                spl = plsc.load_gather(sv, [zi16, jnp.full((16,), i, jnp.int32)])
                for cc in range(C // 16):
                    sl = pl.ds(cc * 16, 16)
                    rows[i, sl] = rows[i, sl] * spl

            pltpu.sync_copy(rows, acc_sh.at[dstv.at[0]], add=True)

        plsc.subcore_barrier()

        base = pl.multiple_of(sid * ROWS_PER_TILE, 8)
        pltpu.sync_copy(acc_sh.at[pl.ds(base, ROWS_PER_TILE)],
                        acc_hbm.at[cid, pl.ds(base, ROWS_PER_TILE)])

        @pl.when(sid == NS - 1)
        def _():
            pltpu.sync_copy(acc_sh.at[pl.ds(TAIL_BASE, TAIL)],
                            acc_hbm.at[cid, pl.ds(TAIL_BASE, TAIL)])

    return k(h, src, dst, s)


def _tc_post_body(acc0, acc1, den_ref, h_ref, a_ref, b_ref, out_ref):
    al = a_ref[:, 0:1] + a_ref[:, 1:2]
    al = jnp.where(al >= 0.0, al, al * 0.2)
    sii = jnp.exp(al)
    num = acc0[...] + acc1[...] + sii * h_ref[...]
    den = jnp.sum(den_ref[...], axis=1, keepdims=True) + sii + 1e-16
    o = num / den + b_ref[...]
    out_ref[...] = jnp.where(o > 0.0, o, jnp.exp(o) - 1.0)


def _tc_post(acc, den, h, a, bias):
    grid = (N // _R,)
    return pl.pallas_call(
        _tc_post_body,
        grid=grid,
        in_specs=[
            pl.BlockSpec((_R, C), lambda i: (i, 0)),
            pl.BlockSpec((_R, C), lambda i: (i, 0)),
            pl.BlockSpec((_R, NW), lambda i: (i, 0)),
            pl.BlockSpec((_R, C), lambda i: (i, 0)),
            pl.BlockSpec((_R, C), lambda i: (i, 0)),
            pl.BlockSpec((1, C), lambda i: (0, 0)),
        ],
        out_specs=pl.BlockSpec((_R, C), lambda i: (i, 0)),
        out_shape=jax.ShapeDtypeStruct((N, C), jnp.float32),
    )(acc[0], acc[1], den, h, a, bias)


def kernel(x, edge_index, W, att_src, att_dst, bias):
    a2 = jnp.zeros((C, C), jnp.float32)
    a2 = a2.at[:, 0].set(att_src[0]).at[:, 1].set(att_dst[0])
    h, a = _tc_pre(x, W, a2)

    asrc = a[:, 0]
    adst = a[:, 1]
    src = edge_index[0].reshape(NW * NCHUNK, 1, B)
    dst = edge_index[1].reshape(NW * NCHUNK, 1, B)

    s, den = _sc_logits(asrc, adst, src, dst)
    acc = _sc_messages(h, src, dst, s)
    den_t = den.reshape(NW, N).T
    return _tc_post(acc, den_t, h, a, bias.reshape(1, C))


# speedup vs baseline: 20.3973x; 1.2172x over previous
import dataclasses
import functools

import jax
import jax.numpy as jnp
from jax import lax
from jax.experimental import pallas as pl
from jax.experimental.pallas import tpu as pltpu
from jax.experimental.pallas import tpu_sc as plsc

N = 10000
E = 320000
C = 128
NC = 2
NS = 16
NW = NC * NS
EPW = E // NW
B = 80
NCHUNK = EPW // B
ROWS_PER_TILE = 624
TAIL_BASE = ROWS_PER_TILE * NS
TAIL = N - TAIL_BASE

_R = 400


def _tc_pre_body(x_ref, w_ref, a2_ref, h_ref, a_ref):
    h = jnp.dot(x_ref[...], w_ref[...], precision=lax.Precision.HIGHEST,
                preferred_element_type=jnp.float32)
    h_ref[...] = h
    a_ref[...] = jnp.dot(h, a2_ref[...], precision=lax.Precision.HIGHEST,
                         preferred_element_type=jnp.float32)


def _tc_pre(x, W, a2):
    grid = (N // _R,)
    return pl.pallas_call(
        _tc_pre_body,
        grid=grid,
        in_specs=[
            pl.BlockSpec((_R, C), lambda i: (i, 0)),
            pl.BlockSpec((C, C), lambda i: (0, 0)),
            pl.BlockSpec((C, C), lambda i: (0, 0)),
        ],
        out_specs=[
            pl.BlockSpec((_R, C), lambda i: (i, 0)),
            pl.BlockSpec((_R, C), lambda i: (i, 0)),
        ],
        out_shape=[
            jax.ShapeDtypeStruct((N, C), jnp.float32),
            jax.ShapeDtypeStruct((N, C), jnp.float32),
        ],
    )(x, W, a2)


def _sc_mesh_and_params():
    mesh = plsc.VectorSubcoreMesh(core_axis_name="c", subcore_axis_name="s")
    cp = pltpu.CompilerParams()
    if "needs_layout_passes" in pltpu.CompilerParams.__dataclass_fields__:
        cp = dataclasses.replace(cp, needs_layout_passes=False)
    return mesh, cp


def _sc_logits(asrc, adst, src, dst):
    mesh, cp = _sc_mesh_and_params()

    @functools.partial(
        pl.kernel,
        compiler_params=cp,
        out_type=[
            jax.ShapeDtypeStruct((NW * NCHUNK, 1, B), jnp.float32),
            jax.ShapeDtypeStruct((NW, 1, N), jnp.float32),
        ],
        mesh=mesh,
        scratch_types=[
            pltpu.VMEM((N,), jnp.float32),
            pltpu.VMEM((N,), jnp.float32),
            pltpu.VMEM((1, N), jnp.float32),
            pltpu.VMEM((1, B), jnp.int32),
            pltpu.VMEM((1, B), jnp.int32),
            pltpu.VMEM((1, B), jnp.float32),
        ],
    )
    def k(asrc_hbm, adst_hbm, src_hbm, dst_hbm, s_hbm, den_hbm,
          asrc_v, adst_v, den_v, srcv, dstv, sv):
        cid = lax.axis_index("c")
        sid = lax.axis_index("s")
        wid = sid * NC + cid

        zero16 = jnp.zeros((16,), jnp.float32)

        zi16 = jnp.zeros((16,), jnp.int32)

        @pl.loop(0, N, step=16)
        def _(r):
            den_v[0, pl.ds(r, 16)] = zero16

        pltpu.sync_copy(asrc_hbm, asrc_v)
        pltpu.sync_copy(adst_hbm, adst_v)

        @pl.loop(0, NCHUNK)
        def _(g):
            row = wid * NCHUNK + g
            pltpu.sync_copy(src_hbm.at[row], srcv)
            pltpu.sync_copy(dst_hbm.at[row], dstv)

            @pl.loop(0, B, step=16)
            def _(i):
                si = srcv[0, pl.ds(i, 16)]
                di = dstv[0, pl.ds(i, 16)]
                al = plsc.load_gather(asrc_v, [si]) + plsc.load_gather(adst_v, [di])
                al = jnp.where(al >= 0.0, al, al * 0.2)
                sval = jnp.exp(al)
                sv[0, pl.ds(i, 16)] = sval
                plsc.addupdate_scatter(den_v, [zi16, di], sval)

            pltpu.sync_copy(sv, s_hbm.at[row])

        pltpu.sync_copy(den_v, den_hbm.at[wid])

    return k(asrc, adst, src, dst)


def _sc_messages(h, src, dst, s):
    mesh, cp = _sc_mesh_and_params()

    @functools.partial(
        pl.kernel,
        compiler_params=cp,
        out_type=jax.ShapeDtypeStruct((NC, N, C), jnp.float32),
        mesh=mesh,
        scratch_types=[
            pltpu.VMEM((2, 1, B), jnp.int32),
            pltpu.VMEM((2, 1, B), jnp.int32),
            pltpu.VMEM((2, 1, B), jnp.float32),
            pltpu.VMEM((2, B, C), jnp.float32),
            pltpu.VMEM_SHARED((N, C), jnp.float32),
            pltpu.SemaphoreType.DMA,
            pltpu.SemaphoreType.DMA,
        ],
    )
    def k(h_hbm, src_hbm, dst_hbm, s_hbm, acc_hbm,
          srcv2, dstv2, sv2, rows2, acc_sh, sem0, sem1):
        cid = lax.axis_index("c")
        sid = lax.axis_index("s")
        wid = sid * NC + cid

        zero16 = jnp.zeros((16,), jnp.float32)
        bufs = [(srcv2.at[b], dstv2.at[b], sv2.at[b], rows2.at[b],
                 (sem0, sem1)[b]) for b in range(2)]

        ZB = 80
        zrows = rows2.at[0]

        @pl.loop(0, ZB)
        def _(r):
            for cc in range(C // 16):
                zrows[r, pl.ds(cc * 16, 16)] = zero16

        @pl.loop(0, ROWS_PER_TILE // ZB)
        def _(z):
            base = pl.multiple_of(sid * ROWS_PER_TILE + z * ZB, 8)
            pltpu.sync_copy(zrows.at[pl.ds(0, ZB)], acc_sh.at[pl.ds(base, ZB)])
        pltpu.sync_copy(zrows.at[pl.ds(0, ROWS_PER_TILE % ZB)],
                        acc_sh.at[pl.ds(pl.multiple_of(
                            sid * ROWS_PER_TILE + (ROWS_PER_TILE // ZB) * ZB, 8),
                            ROWS_PER_TILE % ZB)])

        @pl.when(sid == NS - 1)
        def _():
            pltpu.sync_copy(zrows.at[pl.ds(0, TAIL)],
                            acc_sh.at[pl.ds(TAIL_BASE, TAIL)])

        plsc.subcore_barrier()

        zi16 = jnp.zeros((16,), jnp.int32)

        def fetch(b, g):
            srcv, dstv, sv, rows, sem = bufs[b]
            row = wid * NCHUNK + g
            pltpu.sync_copy(src_hbm.at[row], srcv)
            pltpu.sync_copy(dst_hbm.at[row], dstv)
            pltpu.sync_copy(s_hbm.at[row], sv)
            pltpu.async_copy(h_hbm.at[srcv.at[0]], rows, sem)

        def process(b):
            srcv, dstv, sv, rows, sem = bufs[b]
            pltpu.make_async_copy(h_hbm.at[srcv.at[0]], rows, sem).wait()

            @pl.loop(0, B)
            def _(i):
                spl = plsc.load_gather(sv, [zi16, jnp.full((16,), i, jnp.int32)])
                for cc in range(C // 16):
                    sl = pl.ds(cc * 16, 16)
                    rows[i, sl] = rows[i, sl] * spl

            pltpu.sync_copy(rows, acc_sh.at[dstv.at[0]], add=True)

        fetch(0, 0)

        @pl.loop(0, (NCHUNK - 1) // 2)
        def _(k):
            g = k * 2
            fetch(1, g + 1)
            process(0)
            fetch(0, g + 2)
            process(1)

        process(0)

        plsc.subcore_barrier()

        base = pl.multiple_of(sid * ROWS_PER_TILE, 8)
        pltpu.sync_copy(acc_sh.at[pl.ds(base, ROWS_PER_TILE)],
                        acc_hbm.at[cid, pl.ds(base, ROWS_PER_TILE)])

        @pl.when(sid == NS - 1)
        def _():
            pltpu.sync_copy(acc_sh.at[pl.ds(TAIL_BASE, TAIL)],
                            acc_hbm.at[cid, pl.ds(TAIL_BASE, TAIL)])

    return k(h, src, dst, s)


def _tc_post_body(acc0, acc1, den_ref, h_ref, a_ref, b_ref, out_ref):
    al = a_ref[:, 0:1] + a_ref[:, 1:2]
    al = jnp.where(al >= 0.0, al, al * 0.2)
    sii = jnp.exp(al)
    num = acc0[...] + acc1[...] + sii * h_ref[...]
    den = jnp.sum(den_ref[...], axis=1, keepdims=True) + sii + 1e-16
    o = num / den + b_ref[...]
    out_ref[...] = jnp.where(o > 0.0, o, jnp.exp(o) - 1.0)


def _tc_post(acc, den, h, a, bias):
    grid = (N // _R,)
    return pl.pallas_call(
        _tc_post_body,
        grid=grid,
        in_specs=[
            pl.BlockSpec((_R, C), lambda i: (i, 0)),
            pl.BlockSpec((_R, C), lambda i: (i, 0)),
            pl.BlockSpec((_R, NW), lambda i: (i, 0)),
            pl.BlockSpec((_R, C), lambda i: (i, 0)),
            pl.BlockSpec((_R, C), lambda i: (i, 0)),
            pl.BlockSpec((1, C), lambda i: (0, 0)),
        ],
        out_specs=pl.BlockSpec((_R, C), lambda i: (i, 0)),
        out_shape=jax.ShapeDtypeStruct((N, C), jnp.float32),
    )(acc[0], acc[1], den, h, a, bias)


def kernel(x, edge_index, W, att_src, att_dst, bias):
    a2 = jnp.zeros((C, C), jnp.float32)
    a2 = a2.at[:, 0].set(att_src[0]).at[:, 1].set(att_dst[0])
    h, a = _tc_pre(x, W, a2)

    asrc = a[:, 0]
    adst = a[:, 1]
    src = edge_index[0].reshape(NW * NCHUNK, 1, B)
    dst = edge_index[1].reshape(NW * NCHUNK, 1, B)

    s, den = _sc_logits(asrc, adst, src, dst)
    acc = _sc_messages(h, src, dst, s)
    den_t = den.reshape(NW, N).T
    return _tc_post(acc, den_t, h, a, bias.reshape(1, C))


# speedup vs baseline: 25.3015x; 1.2404x over previous
import dataclasses
import functools

import jax
import jax.numpy as jnp
from jax import lax
from jax.experimental import pallas as pl
from jax.experimental.pallas import tpu as pltpu
from jax.experimental.pallas import tpu_sc as plsc

N = 10000
E = 320000
C = 128
NC = 2
NS = 16
NW = NC * NS
EPW = E // NW
B = 80
NCHUNK = EPW // B
ROWS_PER_TILE = 624
TAIL_BASE = ROWS_PER_TILE * NS
TAIL = N - TAIL_BASE

_R = 400


def _tc_pre_body(x_ref, w_ref, a2_ref, h_ref, a_ref):
    h = jnp.dot(x_ref[...], w_ref[...], precision=lax.Precision.HIGHEST,
                preferred_element_type=jnp.float32)
    h_ref[...] = h
    a_ref[...] = jnp.dot(h, a2_ref[...], precision=lax.Precision.HIGHEST,
                         preferred_element_type=jnp.float32)


def _tc_pre(x, W, a2):
    grid = (N // _R,)
    return pl.pallas_call(
        _tc_pre_body,
        grid=grid,
        in_specs=[
            pl.BlockSpec((_R, C), lambda i: (i, 0)),
            pl.BlockSpec((C, C), lambda i: (0, 0)),
            pl.BlockSpec((C, C), lambda i: (0, 0)),
        ],
        out_specs=[
            pl.BlockSpec((_R, C), lambda i: (i, 0)),
            pl.BlockSpec((_R, C), lambda i: (i, 0)),
        ],
        out_shape=[
            jax.ShapeDtypeStruct((N, C), jnp.float32),
            jax.ShapeDtypeStruct((N, C), jnp.float32),
        ],
    )(x, W, a2)


def _sc_mesh_and_params():
    mesh = plsc.VectorSubcoreMesh(core_axis_name="c", subcore_axis_name="s")
    cp = pltpu.CompilerParams()
    if "needs_layout_passes" in pltpu.CompilerParams.__dataclass_fields__:
        cp = dataclasses.replace(cp, needs_layout_passes=False)
    return mesh, cp


def _sc_logits(asrc, adst, src, dst):
    mesh, cp = _sc_mesh_and_params()

    @functools.partial(
        pl.kernel,
        compiler_params=cp,
        out_type=[
            jax.ShapeDtypeStruct((NW, 1, EPW), jnp.float32),
            jax.ShapeDtypeStruct((NW, 1, N), jnp.float32),
        ],
        mesh=mesh,
        scratch_types=[
            pltpu.VMEM((N,), jnp.float32),
            pltpu.VMEM((N,), jnp.float32),
            pltpu.VMEM((1, N), jnp.float32),
            pltpu.VMEM((1, EPW), jnp.int32),
            pltpu.VMEM((1, EPW), jnp.int32),
            pltpu.VMEM((1, EPW), jnp.float32),
        ],
    )
    def k(asrc_hbm, adst_hbm, src_hbm, dst_hbm, s_hbm, den_hbm,
          asrc_v, adst_v, den_v, srcv, dstv, sv):
        cid = lax.axis_index("c")
        sid = lax.axis_index("s")
        wid = sid * NC + cid

        zero16 = jnp.zeros((16,), jnp.float32)
        zi16 = jnp.zeros((16,), jnp.int32)

        @pl.loop(0, N, step=16)
        def _(r):
            den_v[0, pl.ds(r, 16)] = zero16

        pltpu.sync_copy(asrc_hbm, asrc_v)
        pltpu.sync_copy(adst_hbm, adst_v)
        pltpu.sync_copy(src_hbm.at[wid], srcv)
        pltpu.sync_copy(dst_hbm.at[wid], dstv)

        @pl.loop(0, EPW, step=16)
        def _(i):
            si = srcv[0, pl.ds(i, 16)]
            di = dstv[0, pl.ds(i, 16)]
            al = plsc.load_gather(asrc_v, [si]) + plsc.load_gather(adst_v, [di])
            al = jnp.where(al >= 0.0, al, al * 0.2)
            sval = jnp.exp(al)
            sv[0, pl.ds(i, 16)] = sval
            plsc.addupdate_scatter(den_v, [zi16, di], sval)

        pltpu.sync_copy(sv, s_hbm.at[wid])

        pltpu.sync_copy(den_v, den_hbm.at[wid])

    return k(asrc, adst, src, dst)


def _sc_messages(h, src, dst, s):
    mesh, cp = _sc_mesh_and_params()

    @functools.partial(
        pl.kernel,
        compiler_params=cp,
        out_type=jax.ShapeDtypeStruct((NC, N, C), jnp.float32),
        mesh=mesh,
        scratch_types=[
            pltpu.VMEM((2, 1, B), jnp.int32),
            pltpu.VMEM((2, 1, B), jnp.int32),
            pltpu.VMEM((2, 1, B), jnp.float32),
            pltpu.VMEM((2, B, C), jnp.float32),
            pltpu.VMEM_SHARED((N, C), jnp.float32),
            pltpu.SemaphoreType.DMA,
            pltpu.SemaphoreType.DMA,
        ],
    )
    def k(h_hbm, src_hbm, dst_hbm, s_hbm, acc_hbm,
          srcv2, dstv2, sv2, rows2, acc_sh, sem0, sem1):
        cid = lax.axis_index("c")
        sid = lax.axis_index("s")
        wid = sid * NC + cid

        zero16 = jnp.zeros((16,), jnp.float32)
        bufs = [(srcv2.at[b], dstv2.at[b], sv2.at[b], rows2.at[b],
                 (sem0, sem1)[b]) for b in range(2)]

        ZB = 80
        zrows = rows2.at[0]

        @pl.loop(0, ZB)
        def _(r):
            for cc in range(C // 16):
                zrows[r, pl.ds(cc * 16, 16)] = zero16

        @pl.loop(0, ROWS_PER_TILE // ZB)
        def _(z):
            base = pl.multiple_of(sid * ROWS_PER_TILE + z * ZB, 8)
            pltpu.sync_copy(zrows.at[pl.ds(0, ZB)], acc_sh.at[pl.ds(base, ZB)])
        pltpu.sync_copy(zrows.at[pl.ds(0, ROWS_PER_TILE % ZB)],
                        acc_sh.at[pl.ds(pl.multiple_of(
                            sid * ROWS_PER_TILE + (ROWS_PER_TILE // ZB) * ZB, 8),
                            ROWS_PER_TILE % ZB)])

        @pl.when(sid == NS - 1)
        def _():
            pltpu.sync_copy(zrows.at[pl.ds(0, TAIL)],
                            acc_sh.at[pl.ds(TAIL_BASE, TAIL)])

        plsc.subcore_barrier()

        zi16 = jnp.zeros((16,), jnp.int32)

        def fetch(b, g):
            srcv, dstv, sv, rows, sem = bufs[b]
            row = wid * NCHUNK + g
            pltpu.sync_copy(src_hbm.at[row], srcv)
            pltpu.sync_copy(dst_hbm.at[row], dstv)
            pltpu.sync_copy(s_hbm.at[row], sv)
            pltpu.async_copy(h_hbm.at[srcv.at[0]], rows, sem)

        def process(b):
            srcv, dstv, sv, rows, sem = bufs[b]
            pltpu.make_async_copy(h_hbm.at[srcv.at[0]], rows, sem).wait()

            @pl.loop(0, B)
            def _(i):
                spl = plsc.load_gather(sv, [zi16, jnp.full((16,), i, jnp.int32)])
                for cc in range(C // 16):
                    sl = pl.ds(cc * 16, 16)
                    rows[i, sl] = rows[i, sl] * spl

            pltpu.sync_copy(rows, acc_sh.at[dstv.at[0]], add=True)

        fetch(0, 0)

        @pl.loop(0, (NCHUNK - 1) // 2)
        def _(k):
            g = k * 2
            fetch(1, g + 1)
            process(0)
            fetch(0, g + 2)
            process(1)

        process(0)

        plsc.subcore_barrier()

        base = pl.multiple_of(sid * ROWS_PER_TILE, 8)
        pltpu.sync_copy(acc_sh.at[pl.ds(base, ROWS_PER_TILE)],
                        acc_hbm.at[cid, pl.ds(base, ROWS_PER_TILE)])

        @pl.when(sid == NS - 1)
        def _():
            pltpu.sync_copy(acc_sh.at[pl.ds(TAIL_BASE, TAIL)],
                            acc_hbm.at[cid, pl.ds(TAIL_BASE, TAIL)])

    return k(h, src, dst, s)


def _tc_post_body(acc0, acc1, den_ref, h_ref, a_ref, b_ref, out_ref):
    al = a_ref[:, 0:1] + a_ref[:, 1:2]
    al = jnp.where(al >= 0.0, al, al * 0.2)
    sii = jnp.exp(al)
    num = acc0[...] + acc1[...] + sii * h_ref[...]
    den = jnp.sum(den_ref[...], axis=1, keepdims=True) + sii + 1e-16
    o = num / den + b_ref[...]
    out_ref[...] = jnp.where(o > 0.0, o, jnp.exp(o) - 1.0)


def _tc_post(acc, den, h, a, bias):
    grid = (N // _R,)
    return pl.pallas_call(
        _tc_post_body,
        grid=grid,
        in_specs=[
            pl.BlockSpec((_R, C), lambda i: (i, 0)),
            pl.BlockSpec((_R, C), lambda i: (i, 0)),
            pl.BlockSpec((_R, NW), lambda i: (i, 0)),
            pl.BlockSpec((_R, C), lambda i: (i, 0)),
            pl.BlockSpec((_R, C), lambda i: (i, 0)),
            pl.BlockSpec((1, C), lambda i: (0, 0)),
        ],
        out_specs=pl.BlockSpec((_R, C), lambda i: (i, 0)),
        out_shape=jax.ShapeDtypeStruct((N, C), jnp.float32),
    )(acc[0], acc[1], den, h, a, bias)


def kernel(x, edge_index, W, att_src, att_dst, bias):
    a2 = jnp.zeros((C, C), jnp.float32)
    a2 = a2.at[:, 0].set(att_src[0]).at[:, 1].set(att_dst[0])
    h, a = _tc_pre(x, W, a2)

    asrc = a[:, 0]
    adst = a[:, 1]
    src = edge_index[0].reshape(NW * NCHUNK, 1, B)
    dst = edge_index[1].reshape(NW * NCHUNK, 1, B)
    src_a = edge_index[0].reshape(NW, 1, EPW)
    dst_a = edge_index[1].reshape(NW, 1, EPW)

    s, den = _sc_logits(asrc, adst, src_a, dst_a)
    acc = _sc_messages(h, src, dst, s.reshape(NW * NCHUNK, 1, B))
    den_t = den.reshape(NW, N).T
    return _tc_post(acc, den_t, h, a, bias.reshape(1, C))
